# baseline (speedup 1.0000x reference)
import functools

import jax
import jax.numpy as jnp
from jax import lax
from jax.experimental import pallas as pl
from jax.experimental.pallas import tpu as pltpu
from jax.experimental.pallas import tpu_sc as plsc

_TEMP_INV = 2.0
_THRESH = 0.999
_L = 16


def _make_kernel(n_rows: int, n_cols: int):
    n_vec = n_cols // _L
    mesh = plsc.VectorSubcoreMesh(core_axis_name="c", subcore_axis_name="s")

    @functools.partial(
        pl.kernel,
        out_type=jax.ShapeDtypeStruct((n_rows, n_cols), jnp.float32),
        mesh=mesh,
        compiler_params=pltpu.CompilerParams(needs_layout_passes=False),
        scratch_types=[
            pltpu.VMEM((n_cols,), jnp.float32),
            pltpu.VMEM((n_cols,), jnp.float32),
        ],
    )
    def sharpen_kernel(x_hbm, out_hbm, x_v, y_v):
        cid = lax.axis_index("c")
        sid = lax.axis_index("s")
        wid = sid * 2 + cid

        pltpu.sync_copy(x_hbm.at[wid], x_v)

        zeros = jnp.zeros((_L,), jnp.float32)
        last = jnp.full((_L, 1), _L - 1, jnp.int32)
        dnums = lax.GatherDimensionNumbers(
            offset_dims=(), collapsed_slice_dims=(0,), start_index_map=(0,))

        def bcast_last(vec):
            return lax.gather(
                vec, last, dnums, slice_sizes=(1,),
                mode=lax.GatherScatterMode.PROMISE_IN_BOUNDS)

        def bcast_sum(vec):
            return bcast_last(plsc.cumsum(vec))

        def bcast_max(vec):
            return bcast_last(plsc.cummax(vec))

        def pass1(i, carry):
            acc_s0, acc_sall, acc_nb = carry
            v = x_v[pl.ds(i * _L, _L)]
            p = v * v
            above = v > _THRESH
            acc_s0 = acc_s0 + jnp.where(above, 0.0, p)
            acc_sall = acc_sall + p
            acc_nb = acc_nb + jnp.where(above, 1.0, 0.0)
            return acc_s0, acc_sall, acc_nb

        acc_s0, acc_sall, acc_nb = lax.fori_loop(
            0, n_vec, pass1, (zeros, zeros, zeros))
        s0_vec = bcast_sum(acc_s0)
        sall_vec = bcast_sum(acc_sall)
        nb_vec = bcast_sum(acc_nb)
        sa_vec = sall_vec - s0_vec

        use_fast = jnp.any(s0_vec > 1000.0)

        def approx_sum_inv():
            return (nb_vec - sa_vec / s0_vec) / s0_vec

        def exact_sum_inv():
            def pass2(i, acc):
                v = x_v[pl.ds(i * _L, _L)]
                p = v * v
                above = v > _THRESH
                return acc + jnp.where(above, 1.0 / (s0_vec + p), 0.0)

            return bcast_sum(lax.fori_loop(0, n_vec, pass2, zeros))

        sum_inv_vec = lax.cond(use_fast, approx_sum_inv, exact_sum_inv)

        c_vec = sum_inv_vec / jnp.maximum(nb_vec, 1.0)

        @pl.when(use_fast)
        def _():
            r1_vec = 1.0 / s0_vec

            def pass3_fast(i, carry):
                v = x_v[pl.ds(i * _L, _L)]
                p = v * v
                above = v > _THRESH
                inv = r1_vec - (p * r1_vec) * r1_vec
                w = jnp.where(above, inv, c_vec)
                y_v[pl.ds(i * _L, _L)] = p * w
                return carry

            lax.fori_loop(0, n_vec, pass3_fast, 0)

        @pl.when(jnp.logical_not(use_fast))
        def _():
            def pass3_exact(i, carry):
                v = x_v[pl.ds(i * _L, _L)]
                p = v * v
                above = v > _THRESH
                r = jnp.where(above, p / (s0_vec + p), p * c_vec)
                y_v[pl.ds(i * _L, _L)] = r
                return carry

            lax.fori_loop(0, n_vec, pass3_exact, 0)

        any_above_row = jnp.any(nb_vec > 0.0)

        @pl.when(any_above_row)
        def _():
            pltpu.sync_copy(y_v, out_hbm.at[wid])

        @pl.when(jnp.logical_not(any_above_row))
        def _():
            pltpu.sync_copy(x_v, out_hbm.at[wid])

    return sharpen_kernel


@jax.jit
def _sharpen_multi(batch):
    n_rows, n_cols = batch.shape
    return _make_kernel(n_rows, n_cols)(batch)


def kernel(batch, dim):
    return _sharpen_multi(batch)


# speedup vs baseline: 1.3641x; 1.3641x over previous
import functools

import jax
import jax.numpy as jnp
from jax import lax
from jax.experimental import pallas as pl
from jax.experimental.pallas import tpu as pltpu
from jax.experimental.pallas import tpu_sc as plsc

_TEMP_INV = 2.0
_THRESH = 0.999
_L = 16


def _make_kernel(n_rows: int, n_cols: int):
    n_vec = n_cols // _L
    mesh = plsc.VectorSubcoreMesh(core_axis_name="c", subcore_axis_name="s")

    @functools.partial(
        pl.kernel,
        out_type=jax.ShapeDtypeStruct((n_rows, n_cols), jnp.float32),
        mesh=mesh,
        compiler_params=pltpu.CompilerParams(needs_layout_passes=False),
        scratch_types=[
            pltpu.VMEM((n_cols,), jnp.float32),
            pltpu.VMEM((n_cols,), jnp.float32),
        ],
    )
    def sharpen_kernel(x_hbm, out_hbm, x_v, y_v):
        cid = lax.axis_index("c")
        sid = lax.axis_index("s")
        wid = sid * 2 + cid

        pltpu.sync_copy(x_hbm.at[wid], x_v)

        zeros = jnp.zeros((_L,), jnp.float32)
        last = jnp.full((_L, 1), _L - 1, jnp.int32)
        dnums = lax.GatherDimensionNumbers(
            offset_dims=(), collapsed_slice_dims=(0,), start_index_map=(0,))

        def bcast_last(vec):
            return lax.gather(
                vec, last, dnums, slice_sizes=(1,),
                mode=lax.GatherScatterMode.PROMISE_IN_BOUNDS)

        def bcast_sum(vec):
            return bcast_last(plsc.cumsum(vec))

        def bcast_max(vec):
            return bcast_last(plsc.cummax(vec))

        def pass1(i, carry):
            acc_s0, acc_sall, acc_nb = carry
            v = x_v[pl.ds(i * _L, _L)]
            p = v * v
            above = v > _THRESH
            acc_s0 = acc_s0 + jnp.where(above, 0.0, p)
            acc_sall = acc_sall + p
            acc_nb = acc_nb + jnp.where(above, 1.0, 0.0)
            return acc_s0, acc_sall, acc_nb

        acc_s0, acc_sall, acc_nb = plsc.parallel_loop(
            0, n_vec, unroll=4, carry=(zeros, zeros, zeros))(
                lambda i, carry: pass1(i, carry))
        s0_vec = bcast_sum(acc_s0)
        sall_vec = bcast_sum(acc_sall)
        nb_vec = bcast_sum(acc_nb)
        sa_vec = sall_vec - s0_vec

        use_fast = jnp.any(s0_vec > 1000.0)

        def approx_sum_inv():
            return (nb_vec - sa_vec / s0_vec) / s0_vec

        def exact_sum_inv():
            def pass2(i, acc):
                v = x_v[pl.ds(i * _L, _L)]
                p = v * v
                above = v > _THRESH
                return acc + jnp.where(above, 1.0 / (s0_vec + p), 0.0)

            return bcast_sum(lax.fori_loop(0, n_vec, pass2, zeros))

        sum_inv_vec = lax.cond(use_fast, approx_sum_inv, exact_sum_inv)

        c_vec = sum_inv_vec / jnp.maximum(nb_vec, 1.0)

        @pl.when(use_fast)
        def _():
            r1_vec = 1.0 / s0_vec

            @plsc.parallel_loop(0, n_vec, unroll=4)
            def pass3_fast(i):
                v = x_v[pl.ds(i * _L, _L)]
                p = v * v
                above = v > _THRESH
                inv = r1_vec - (p * r1_vec) * r1_vec
                w = jnp.where(above, inv, c_vec)
                y_v[pl.ds(i * _L, _L)] = p * w

        @pl.when(jnp.logical_not(use_fast))
        def _():
            def pass3_exact(i, carry):
                v = x_v[pl.ds(i * _L, _L)]
                p = v * v
                above = v > _THRESH
                r = jnp.where(above, p / (s0_vec + p), p * c_vec)
                y_v[pl.ds(i * _L, _L)] = r
                return carry

            lax.fori_loop(0, n_vec, pass3_exact, 0)

        any_above_row = jnp.any(nb_vec > 0.0)

        @pl.when(any_above_row)
        def _():
            pltpu.sync_copy(y_v, out_hbm.at[wid])

        @pl.when(jnp.logical_not(any_above_row))
        def _():
            pltpu.sync_copy(x_v, out_hbm.at[wid])

    return sharpen_kernel


@jax.jit
def _sharpen_multi(batch):
    n_rows, n_cols = batch.shape
    return _make_kernel(n_rows, n_cols)(batch)


def kernel(batch, dim):
    return _sharpen_multi(batch)


# speedup vs baseline: 1.3726x; 1.0063x over previous
import functools

import jax
import jax.numpy as jnp
from jax import lax
from jax.experimental import pallas as pl
from jax.experimental.pallas import tpu as pltpu
from jax.experimental.pallas import tpu_sc as plsc

_TEMP_INV = 2.0
_THRESH = 0.999
_L = 16


def _make_kernel(n_rows: int, n_cols: int):
    n_vec = n_cols // _L
    mesh = plsc.VectorSubcoreMesh(core_axis_name="c", subcore_axis_name="s")

    @functools.partial(
        pl.kernel,
        out_type=jax.ShapeDtypeStruct((n_rows, n_cols), jnp.float32),
        mesh=mesh,
        compiler_params=pltpu.CompilerParams(needs_layout_passes=False),
        scratch_types=[
            pltpu.VMEM((n_cols,), jnp.float32),
            pltpu.VMEM((n_cols,), jnp.float32),
        ],
    )
    def sharpen_kernel(x_hbm, out_hbm, x_v, y_v):
        cid = lax.axis_index("c")
        sid = lax.axis_index("s")
        wid = sid * 2 + cid

        pltpu.sync_copy(x_hbm.at[wid], x_v)

        zeros = jnp.zeros((_L,), jnp.float32)
        last = jnp.full((_L, 1), _L - 1, jnp.int32)
        dnums = lax.GatherDimensionNumbers(
            offset_dims=(), collapsed_slice_dims=(0,), start_index_map=(0,))

        def bcast_last(vec):
            return lax.gather(
                vec, last, dnums, slice_sizes=(1,),
                mode=lax.GatherScatterMode.PROMISE_IN_BOUNDS)

        def bcast_sum(vec):
            return bcast_last(plsc.cumsum(vec))

        def bcast_max(vec):
            return bcast_last(plsc.cummax(vec))

        def pass1(i, carry):
            acc_s0, acc_sall, acc_nb = carry
            v = x_v[pl.ds(i * _L, _L)]
            p = v * v
            above = v > _THRESH
            acc_s0 = acc_s0 + jnp.where(above, 0.0, p)
            acc_sall = acc_sall + p
            acc_nb = acc_nb + jnp.where(above, 1.0, 0.0)
            return acc_s0, acc_sall, acc_nb

        acc_s0, acc_sall, acc_nb = plsc.parallel_loop(
            0, n_vec, unroll=8, carry=(zeros, zeros, zeros))(
                lambda i, carry: pass1(i, carry))
        s0_vec = bcast_sum(acc_s0)
        sall_vec = bcast_sum(acc_sall)
        nb_vec = bcast_sum(acc_nb)
        sa_vec = sall_vec - s0_vec

        use_fast = jnp.any(s0_vec > 1000.0)

        def approx_sum_inv():
            return (nb_vec - sa_vec / s0_vec) / s0_vec

        def exact_sum_inv():
            def pass2(i, acc):
                v = x_v[pl.ds(i * _L, _L)]
                p = v * v
                above = v > _THRESH
                return acc + jnp.where(above, 1.0 / (s0_vec + p), 0.0)

            return bcast_sum(lax.fori_loop(0, n_vec, pass2, zeros))

        sum_inv_vec = lax.cond(use_fast, approx_sum_inv, exact_sum_inv)

        c_vec = sum_inv_vec / jnp.maximum(nb_vec, 1.0)

        @pl.when(use_fast)
        def _():
            r1_vec = 1.0 / s0_vec

            @plsc.parallel_loop(0, n_vec, unroll=8)
            def pass3_fast(i):
                v = x_v[pl.ds(i * _L, _L)]
                p = v * v
                above = v > _THRESH
                inv = r1_vec - (p * r1_vec) * r1_vec
                w = jnp.where(above, inv, c_vec)
                y_v[pl.ds(i * _L, _L)] = p * w

        @pl.when(jnp.logical_not(use_fast))
        def _():
            def pass3_exact(i, carry):
                v = x_v[pl.ds(i * _L, _L)]
                p = v * v
                above = v > _THRESH
                r = jnp.where(above, p / (s0_vec + p), p * c_vec)
                y_v[pl.ds(i * _L, _L)] = r
                return carry

            lax.fori_loop(0, n_vec, pass3_exact, 0)

        any_above_row = jnp.any(nb_vec > 0.0)

        @pl.when(any_above_row)
        def _():
            pltpu.sync_copy(y_v, out_hbm.at[wid])

        @pl.when(jnp.logical_not(any_above_row))
        def _():
            pltpu.sync_copy(x_v, out_hbm.at[wid])

    return sharpen_kernel


@jax.jit
def _sharpen_multi(batch):
    n_rows, n_cols = batch.shape
    return _make_kernel(n_rows, n_cols)(batch)


def kernel(batch, dim):
    return _sharpen_multi(batch)
